# baseline (speedup 1.0000x reference)
import functools
import jax
import jax.numpy as jnp
from jax import lax
from jax.experimental import pallas as pl
from jax.experimental.pallas import tpu as pltpu, tpu_sc as plsc

D_MODEL = 64
NC, NS = 2, 16
NW = NC * NS
CG = 8
SG = NW // CG
LANES = 16
NBO = 3


def _gather_body(xT_hbm, peT_hbm, outT_hbm, table_v, idx_v, out_v, osem, isem,
                 *, seqlen, batch, n_c, table_len):
    c = lax.axis_index("c")
    s = lax.axis_index("s")
    wid = s * NC + c
    c0 = (wid % CG) * n_c
    s_per = seqlen // SG
    s0 = (wid // CG) * s_per
    ngroups = batch // LANES
    table_f = table_v

    for cl in range(n_c):
        pltpu.sync_copy(peT_hbm.at[c0 + cl],
                        table_v.at[pl.ds(cl * table_len, table_len)])

    def put(si, b):
        return pltpu.async_copy(
            out_v.at[b], outT_hbm.at[si, pl.ds(c0, n_c)], osem.at[b])

    def wait_put(si, b):
        pltpu.make_async_copy(
            out_v.at[b], outT_hbm.at[si, pl.ds(c0, n_c)], osem.at[b]).wait()

    def get_idx(si, ib):
        return pltpu.async_copy(xT_hbm.at[si], idx_v.at[ib], isem.at[ib])

    def wait_idx(si, ib):
        pltpu.make_async_copy(xT_hbm.at[si], idx_v.at[ib], isem.at[ib]).wait()

    get_idx(s0, 0)

    def step(i, b, ob):
        si = s0 + i + b
        wait_idx(si, b)

        @pl.when(i + b + 1 < s_per)
        def _():
            get_idx(si + 1, 1 - b)

        @pl.when(i + b >= NBO)
        def _():
            wait_put(si - NBO, ob)

        @plsc.parallel_loop(0, batch, step=LANES, unroll=8)
        def _(o):
            idx16 = idx_v[b, pl.ds(o, LANES)]
            for cl in range(n_c):
                val = plsc.load_gather(
                    table_f, [idx16 + jnp.int32(cl * table_len)])
                out_v[ob, cl, pl.ds(o, LANES)] = val

        put(si, ob)

    main = (s_per // (2 * NBO)) * (2 * NBO)

    @pl.loop(0, main, step=2 * NBO)
    def _(i):
        for j in range(2 * NBO):
            step(i + (j // 2) * 2, j % 2, j % NBO)

    for j in range(s_per - main):
        step(main + (j // 2) * 2, j % 2, j % NBO)

    for k in range(NBO):
        si = s0 + s_per - NBO + k
        wait_put(si, (si - s0) % NBO)


@jax.jit
def _pe_gather(xT, peT):
    seqlen, batch = xT.shape
    n_c = peT.shape[0] // CG
    table_len = peT.shape[1]
    mesh = plsc.VectorSubcoreMesh(
        core_axis_name="c", subcore_axis_name="s",
        num_cores=NC, num_subcores=NS)
    body = functools.partial(_gather_body, seqlen=seqlen, batch=batch,
                             n_c=n_c, table_len=table_len)
    k = pl.kernel(
        body,
        out_type=jax.ShapeDtypeStruct((seqlen, peT.shape[0], batch),
                                      jnp.float32),
        mesh=mesh,
        scratch_types=[
            pltpu.VMEM((n_c * table_len,), jnp.float32),
            pltpu.VMEM((2, batch), jnp.int32),
            pltpu.VMEM((NBO, n_c, batch), jnp.float32),
            pltpu.SemaphoreType.DMA((2,)),
            pltpu.SemaphoreType.DMA((2,)),
        ],
        compiler_params=pltpu.CompilerParams(needs_layout_passes=False),
    )
    return k(xT, peT)


def kernel(x, pe):
    xT = jnp.swapaxes(x, 0, 1)
    peT = jnp.swapaxes(pe, 0, 1)
    outT = _pe_gather(xT, peT)
    return outT.transpose(2, 0, 1)


# speedup vs baseline: 1.0199x; 1.0199x over previous
import functools
import jax
import jax.numpy as jnp
from jax import lax
from jax.experimental import pallas as pl
from jax.experimental.pallas import tpu as pltpu, tpu_sc as plsc

D_MODEL = 64
NC, NS = 2, 16
NW = NC * NS
CG = 8
SG = NW // CG
LANES = 16


def _gather_body(xT_hbm, peT_hbm, outT_hbm, table_v, idx_v, out_v, osem, isem,
                 *, seqlen, batch, n_c, table_len):
    c = lax.axis_index("c")
    s = lax.axis_index("s")
    wid = s * NC + c
    c0 = (wid % CG) * n_c
    s_per = seqlen // SG
    s0 = (wid // CG) * s_per
    ngroups = batch // LANES
    table_f = table_v

    for cl in range(n_c):
        pltpu.sync_copy(peT_hbm.at[c0 + cl],
                        table_v.at[pl.ds(cl * table_len, table_len)])

    def put(si, b):
        return pltpu.async_copy(
            out_v.at[b], outT_hbm.at[si, pl.ds(c0, n_c)], osem.at[b])

    def wait_put(si, b):
        pltpu.make_async_copy(
            out_v.at[b], outT_hbm.at[si, pl.ds(c0, n_c)], osem.at[b]).wait()

    def get_idx(si, ib):
        return pltpu.async_copy(xT_hbm.at[si], idx_v.at[ib], isem.at[ib])

    def wait_idx(si, ib):
        pltpu.make_async_copy(xT_hbm.at[si], idx_v.at[ib], isem.at[ib]).wait()

    get_idx(s0, 0)

    @pl.loop(0, s_per, step=2)
    def _(i):
        for b in range(2):
            si = s0 + i + b
            wait_idx(si, b)

            @pl.when(i + b + 1 < s_per)
            def _():
                get_idx(si + 1, 1 - b)

            @pl.when(i + b >= 2)
            def _():
                wait_put(si - 2, b)

            @plsc.parallel_loop(0, batch, step=LANES, unroll=8)
            def _(o):
                idx16 = idx_v[b, pl.ds(o, LANES)]
                for cl in range(n_c):
                    val = plsc.load_gather(
                        table_f, [idx16 + jnp.int32(cl * table_len)])
                    out_v[b, cl, pl.ds(o, LANES)] = val

            put(si, b)

    for b in range(2):
        wait_put(s0 + s_per - 2 + b, b)


@jax.jit
def _pe_gather(xT, peT):
    seqlen, batch = xT.shape
    n_c = peT.shape[0] // CG
    table_len = peT.shape[1]
    mesh = plsc.VectorSubcoreMesh(
        core_axis_name="c", subcore_axis_name="s",
        num_cores=NC, num_subcores=NS)
    body = functools.partial(_gather_body, seqlen=seqlen, batch=batch,
                             n_c=n_c, table_len=table_len)
    k = pl.kernel(
        body,
        out_type=jax.ShapeDtypeStruct((seqlen, peT.shape[0], batch),
                                      jnp.float32),
        mesh=mesh,
        scratch_types=[
            pltpu.VMEM((n_c * table_len,), jnp.float32),
            pltpu.VMEM((2, batch), jnp.int32),
            pltpu.VMEM((2, n_c, batch), jnp.float32),
            pltpu.SemaphoreType.DMA((2,)),
            pltpu.SemaphoreType.DMA((2,)),
        ],
        compiler_params=pltpu.CompilerParams(needs_layout_passes=False),
    )
    return k(xT, peT)


def kernel(x, pe):
    xT = jnp.swapaxes(x, 0, 1)
    peT = jnp.swapaxes(pe, 0, 1)
    outT = _pe_gather(xT, peT)
    return outT.transpose(2, 0, 1)
